# baseline (speedup 1.0000x reference)
import jax
import jax.numpy as jnp
from jax.experimental import pallas as pl


def _add_pos_kernel(x_ref, pos_ref, o_ref):
    o_ref[...] = x_ref[...] + pos_ref[...][:, :, None]


def kernel(x, pos_table):
    B, L, D = x.shape
    xt = jnp.transpose(x, (1, 2, 0))
    LB = 8
    BS = B // 2
    out_t = pl.pallas_call(
        _add_pos_kernel,
        grid=(L // LB, B // BS),
        in_specs=[
            pl.BlockSpec((LB, D, BS), lambda i, j: (i, 0, j)),
            pl.BlockSpec((LB, D), lambda i, j: (i, 0)),
        ],
        out_specs=pl.BlockSpec((LB, D, BS), lambda i, j: (i, 0, j)),
        out_shape=jax.ShapeDtypeStruct((L, D, B), x.dtype),
    )(xt, pos_table)
    return jnp.transpose(out_t, (2, 0, 1))


# speedup vs baseline: 1.0187x; 1.0187x over previous
import jax
import jax.numpy as jnp
from jax.experimental import pallas as pl


def _add_pos_kernel(x_ref, pos_ref, o_ref):
    o_ref[...] = x_ref[...] + pos_ref[...][:, :, None]


def kernel(x, pos_table):
    B, L, D = x.shape
    xt = jnp.transpose(x, (1, 2, 0))
    LB = 8
    out_t = pl.pallas_call(
        _add_pos_kernel,
        grid=(L // LB,),
        in_specs=[
            pl.BlockSpec((LB, D, B), lambda i: (i, 0, 0)),
            pl.BlockSpec((LB, D), lambda i: (i, 0)),
        ],
        out_specs=pl.BlockSpec((LB, D, B), lambda i: (i, 0, 0)),
        out_shape=jax.ShapeDtypeStruct((L, D, B), x.dtype),
    )(xt, pos_table)
    return jnp.transpose(out_t, (2, 0, 1))


# speedup vs baseline: 1.0221x; 1.0033x over previous
import jax
import jax.numpy as jnp
from jax.experimental import pallas as pl


def _add_pos_kernel(x_ref, pos_ref, o_ref):
    o_ref[...] = x_ref[...] + pos_ref[...][:, :, None]


def kernel(x, pos_table):
    B, L, D = x.shape
    xt = jnp.transpose(x, (1, 2, 0))
    pos = jax.lax.slice(pos_table, (0, 0), (L, D))
    LB = 8
    out_t = pl.pallas_call(
        _add_pos_kernel,
        grid=(L // LB,),
        in_specs=[
            pl.BlockSpec((LB, D, B), lambda i: (i, 0, 0)),
            pl.BlockSpec((LB, D), lambda i: (i, 0)),
        ],
        out_specs=pl.BlockSpec((LB, D, B), lambda i: (i, 0, 0)),
        out_shape=jax.ShapeDtypeStruct((L, D, B), x.dtype),
    )(xt, pos)
    return jnp.transpose(out_t, (2, 0, 1))
